# baseline (speedup 1.0000x reference)
import functools
import math

import jax
import jax.numpy as jnp
from jax import lax
from jax.experimental import pallas as pl
from jax.experimental.pallas import tpu as pltpu
from jax.experimental.pallas import tpu_sc as plsc

VOCAB = 100000
D_EMB = 300
D_PAD = 304
SCALE = math.sqrt(float(D_EMB))

NC, NS = 2, 16
NW = NC * NS
CH = 128
LANES = 16
FULL_VREGS = D_EMB // LANES
TAIL = D_EMB - FULL_VREGS * LANES


def _emb_body(x_hbm, table_hbm, out_hbm, idx_v, rows_v, pack_v, gsem, nch):
  wid = lax.axis_index("s") * NC + lax.axis_index("c")
  pltpu.sync_copy(x_hbm.at[wid], idx_v)

  iota = lax.broadcasted_iota(jnp.int32, (LANES,), 0)
  tail_col = FULL_VREGS * LANES + jnp.where(iota < TAIL, iota, TAIL - 1)
  tail_mask = iota < TAIL

  def chunk(i, _):
    pltpu.async_copy(table_hbm.at[idx_v.at[i]], rows_v, gsem).wait()

    def row(r, _):
      for k in range(FULL_VREGS):
        sl = pl.ds(k * LANES, LANES)
        pack_v[r, sl] = rows_v[r, sl] * SCALE
      tail = rows_v[r, pl.ds(FULL_VREGS * LANES, LANES)] * SCALE
      rvec = jnp.zeros((LANES,), jnp.int32) + r
      plsc.store_scatter(pack_v, [rvec, tail_col], tail, mask=tail_mask)
      return 0

    lax.fori_loop(0, CH, row, 0)
    pltpu.sync_copy(pack_v, out_hbm.at[pl.ds((wid * nch + i) * CH, CH)])
    return 0

  lax.fori_loop(0, nch, chunk, 0)


def kernel(x, table):
  B, L = x.shape
  n_total = B * L
  assert n_total % (NW * CH) == 0
  nch = n_total // (NW * CH)

  mesh = plsc.VectorSubcoreMesh(core_axis_name="c", subcore_axis_name="s")
  k = pl.kernel(
      functools.partial(_emb_body, nch=nch),
      out_type=jax.ShapeDtypeStruct((n_total, D_EMB), jnp.float32),
      mesh=mesh,
      compiler_params=pltpu.CompilerParams(
          use_tc_tiling_on_sc=False, needs_layout_passes=False),
      scratch_types=[
          pltpu.VMEM((nch, CH), jnp.int32),
          pltpu.VMEM((CH, D_PAD), jnp.float32),
          pltpu.VMEM((CH, D_EMB), jnp.float32),
          pltpu.SemaphoreType.DMA,
      ],
  )
  x2 = x.reshape(NW, nch, CH)
  table_pad = jnp.pad(table, ((0, 0), (0, D_PAD - D_EMB)))
  out = k(x2, table_pad)
  return out.reshape(B, L, D_EMB)


# speedup vs baseline: 1.0526x; 1.0526x over previous
import functools
import math

import jax
import jax.numpy as jnp
from jax import lax
from jax.experimental import pallas as pl
from jax.experimental.pallas import tpu as pltpu
from jax.experimental.pallas import tpu_sc as plsc

VOCAB = 100000
D_EMB = 300
D_PAD = 304
SCALE = math.sqrt(float(D_EMB))

NC, NS = 2, 16
NW = NC * NS
CH = 128
LANES = 16
VREGS_PER_ROW = D_PAD // LANES
FULL_VREGS = D_EMB // LANES
TAIL = D_EMB - FULL_VREGS * LANES


def _emb_body(x_hbm, table_hbm, out_hbm, idx_v, rows_v, pack_v, gsem, nch):
  wid = lax.axis_index("s") * NC + lax.axis_index("c")
  pltpu.sync_copy(x_hbm.at[wid], idx_v)

  iota = lax.broadcasted_iota(jnp.int32, (LANES,), 0)
  offs = [iota + k * LANES for k in range(FULL_VREGS)]
  offs.append(jnp.where(iota < TAIL, iota, TAIL - 1) + FULL_VREGS * LANES)
  tail_mask = iota < TAIL

  def chunk(i, _):
    pltpu.async_copy(table_hbm.at[idx_v.at[i]], rows_v, gsem).wait()

    def row(r, _):
      base = r * D_EMB
      for k in range(FULL_VREGS):
        v = rows_v[r, pl.ds(k * LANES, LANES)] * SCALE
        plsc.store_scatter(pack_v, [base + offs[k]], v)
      v = rows_v[r, pl.ds(FULL_VREGS * LANES, LANES)] * SCALE
      plsc.store_scatter(pack_v, [base + offs[FULL_VREGS]], v, mask=tail_mask)
      return 0

    lax.fori_loop(0, CH, row, 0)
    pltpu.sync_copy(pack_v, out_hbm.at[pl.ds((wid * nch + i) * CH * D_EMB,
                                             CH * D_EMB)])
    return 0

  lax.fori_loop(0, nch, chunk, 0)


def kernel(x, table):
  B, L = x.shape
  n_total = B * L
  assert n_total % (NW * CH) == 0
  nch = n_total // (NW * CH)

  mesh = plsc.VectorSubcoreMesh(core_axis_name="c", subcore_axis_name="s")
  k = pl.kernel(
      functools.partial(_emb_body, nch=nch),
      out_type=jax.ShapeDtypeStruct((n_total * D_EMB,), jnp.float32),
      mesh=mesh,
      compiler_params=pltpu.CompilerParams(
          use_tc_tiling_on_sc=False, needs_layout_passes=False),
      scratch_types=[
          pltpu.VMEM((nch, CH), jnp.int32),
          pltpu.VMEM((CH, D_PAD), jnp.float32),
          pltpu.VMEM((CH * D_EMB,), jnp.float32),
          pltpu.SemaphoreType.DMA,
      ],
  )
  x2 = x.reshape(NW, nch, CH)
  table_pad = jnp.pad(table, ((0, 0), (0, D_PAD - D_EMB)))
  out = k(x2, table_pad)
  return out.reshape(B, L, D_EMB)


# speedup vs baseline: 1.1533x; 1.0957x over previous
import functools
import math

import jax
import jax.numpy as jnp
from jax import lax
from jax.experimental import pallas as pl
from jax.experimental.pallas import tpu as pltpu
from jax.experimental.pallas import tpu_sc as plsc

VOCAB = 100000
D_EMB = 300
D_PAD = 304
SCALE = math.sqrt(float(D_EMB))

NC, NS = 2, 16
NW = NC * NS
CH = 64
LANES = 16
FULL_VREGS = D_EMB // LANES
TAIL = D_EMB - FULL_VREGS * LANES


def _emb_body(x_hbm, table_hbm, out_hbm, idx_v, rows0, rows1, pack0, pack1,
              g0, g1, w0, w1, nch):
  wid = lax.axis_index("s") * NC + lax.axis_index("c")
  pltpu.sync_copy(x_hbm.at[wid], idx_v)

  iota = lax.broadcasted_iota(jnp.int32, (LANES,), 0)
  offs = [iota + k * LANES for k in range(FULL_VREGS)]
  offs.append(jnp.where(iota < TAIL, iota, TAIL - 1) + FULL_VREGS * LANES)
  tail_mask = iota < TAIL
  rows_bufs = (rows0, rows1)
  pack_bufs = (pack0, pack1)
  g_sems = (g0, g1)
  w_sems = (w0, w1)

  def out_slice(i):
    return out_hbm.at[pl.ds((wid * nch + i) * CH * D_EMB, CH * D_EMB)]

  def scale_chunk(rows_v, pack_v):
    def row(r, _):
      base = r * D_EMB
      for k in range(FULL_VREGS):
        v = rows_v[r, pl.ds(k * LANES, LANES)] * SCALE
        plsc.store_scatter(pack_v, [base + offs[k]], v)
      v = rows_v[r, pl.ds(FULL_VREGS * LANES, LANES)] * SCALE
      plsc.store_scatter(pack_v, [base + offs[FULL_VREGS]], v, mask=tail_mask)
      return 0

    lax.fori_loop(0, CH, row, 0)

  pltpu.async_copy(table_hbm.at[idx_v.at[0]], rows0, g0)

  def pair(j, _):
    for db in range(2):
      i = 2 * j + db
      rows_v, pack_v = rows_bufs[db], pack_bufs[db]
      gs, ws = g_sems[db], w_sems[db]
      nrows, ngs = rows_bufs[1 - db], g_sems[1 - db]

      @pl.when(i + 1 < nch)
      def _():
        pltpu.async_copy(table_hbm.at[idx_v.at[i + 1]], nrows, ngs)

      pltpu.make_async_copy(table_hbm.at[idx_v.at[i]], rows_v, gs).wait()

      @pl.when(i >= 2)
      def _():
        pltpu.make_async_copy(pack_v, out_slice(i - 2), ws).wait()

      scale_chunk(rows_v, pack_v)
      pltpu.async_copy(pack_v, out_slice(i), ws)
    return 0

  lax.fori_loop(0, nch // 2, pair, 0)
  pltpu.make_async_copy(pack0, out_slice(nch - 2), w0).wait()
  pltpu.make_async_copy(pack1, out_slice(nch - 1), w1).wait()


def kernel(x, table):
  B, L = x.shape
  n_total = B * L
  assert n_total % (NW * CH) == 0
  nch = n_total // (NW * CH)

  mesh = plsc.VectorSubcoreMesh(core_axis_name="c", subcore_axis_name="s")
  k = pl.kernel(
      functools.partial(_emb_body, nch=nch),
      out_type=jax.ShapeDtypeStruct((n_total * D_EMB,), jnp.float32),
      mesh=mesh,
      compiler_params=pltpu.CompilerParams(
          use_tc_tiling_on_sc=False, needs_layout_passes=False),
      scratch_types=[
          pltpu.VMEM((nch, CH), jnp.int32),
          pltpu.VMEM((CH, D_PAD), jnp.float32),
          pltpu.VMEM((CH, D_PAD), jnp.float32),
          pltpu.VMEM((CH * D_EMB,), jnp.float32),
          pltpu.VMEM((CH * D_EMB,), jnp.float32),
          pltpu.SemaphoreType.DMA,
          pltpu.SemaphoreType.DMA,
          pltpu.SemaphoreType.DMA,
          pltpu.SemaphoreType.DMA,
      ],
  )
  x2 = x.reshape(NW, nch, CH)
  table_pad = jnp.pad(table, ((0, 0), (0, D_PAD - D_EMB)))
  out = k(x2, table_pad)
  return out.reshape(B, L, D_EMB)
